# baseline (speedup 1.0000x reference)
import jax
import jax.numpy as jnp
import numpy as np
from jax.experimental import pallas as pl
from jax.experimental.pallas import tpu as pltpu

_N_BINS = 20
_N = 1_000_000
_C = 100
_R = 2000
_G = _N // _R

_BOUNDS = [float(v) for v in np.arange(_N_BINS + 1, dtype=np.float32)
           * np.float32(1.0 / _N_BINS)]


def _body(x_ref, lbl_ref, ece_ref, ys_ref, acc_ref):
    pid = pl.program_id(0)

    @pl.when(pid == 0)
    def _init():
        acc_ref[...] = jnp.zeros_like(acc_ref)

    x = x_ref[...]
    lbl = lbl_ref[0, 0, :]
    conf = jnp.max(x, axis=1)
    pred = jnp.argmax(x, axis=1).astype(jnp.int32)
    accuracy = (pred == lbl).astype(jnp.float32)

    bounds = _BOUNDS
    cnts, confs, accs = [], [], []
    for i in range(_N_BINS):
        m = ((conf > bounds[i]) & (conf <= bounds[i + 1])).astype(jnp.float32)
        cnts.append(jnp.sum(m))
        confs.append(jnp.sum(conf * m))
        accs.append(jnp.sum(accuracy * m))
    upd = jnp.stack([jnp.stack(cnts), jnp.stack(confs), jnp.stack(accs)])
    acc_ref[...] += upd

    @pl.when(pid == _G - 1)
    def _fin():
        cnt = acc_ref[0, :]
        sconf = acc_ref[1, :]
        sacc = acc_ref[2, :]
        has = cnt > 0.0
        denom = jnp.maximum(cnt, 1.0)
        acc_in = jnp.where(has, sacc / denom, 0.0)
        conf_in = jnp.where(has, sconf / denom, 0.0)
        prop = cnt * (1.0 / _N)
        ece = jnp.sum(jnp.where(has, jnp.abs(conf_in - acc_in) * prop, 0.0))
        ece_ref[...] = ece.reshape(1, 1)
        ys_ref[...] = acc_in.reshape(1, _N_BINS)


def kernel(logits, labels):
    lbl3 = labels.reshape(_G, 1, _R)
    ece2, ys2 = pl.pallas_call(
        _body,
        grid=(_G,),
        in_specs=[
            pl.BlockSpec((_R, _C), lambda i: (i, 0)),
            pl.BlockSpec((1, 1, _R), lambda i: (i, 0, 0)),
        ],
        out_specs=[
            pl.BlockSpec((1, 1), lambda i: (0, 0)),
            pl.BlockSpec((1, _N_BINS), lambda i: (0, 0)),
        ],
        out_shape=[
            jax.ShapeDtypeStruct((1, 1), jnp.float32),
            jax.ShapeDtypeStruct((1, _N_BINS), jnp.float32),
        ],
        scratch_shapes=[pltpu.VMEM((3, _N_BINS), jnp.float32)],
        compiler_params=pltpu.CompilerParams(
            dimension_semantics=("arbitrary",),
        ),
    )(logits, lbl3)
    return (ece2.reshape(1), ys2.reshape(_N_BINS))


# speedup vs baseline: 4.3140x; 4.3140x over previous
import jax
import jax.numpy as jnp
import numpy as np
from jax import lax
from jax.experimental import pallas as pl
from jax.experimental.pallas import tpu as pltpu

_N_BINS = 20
_N = 1_000_000
_C = 100
_R = 2000
_G = _N // _R

_BOUNDS = np.arange(_N_BINS + 1, dtype=np.float32) * np.float32(1.0 / _N_BINS)

_BROW = np.full((1, 128), np.inf, dtype=np.float32)
_BROW[0, : _N_BINS + 1] = _BOUNDS
_BROW[0, _N_BINS + 1] = -np.inf


def _body(x_ref, lbl_ref, ece_ref, ys_ref, acc_ref):
    pid = pl.program_id(0)

    @pl.when(pid == 0)
    def _init():
        acc_ref[...] = jnp.zeros_like(acc_ref)

    li = lax.broadcasted_iota(jnp.int32, (1, 128), 1)
    brow = jnp.where(li == _N_BINS + 1, -jnp.inf,
                     jnp.where(li >= _N_BINS + 2, jnp.inf,
                               li.astype(jnp.float32)
                               * np.float32(1.0 / _N_BINS)))

    x = x_ref[...]
    conf = jnp.max(x, axis=1, keepdims=True)
    pred = jnp.argmax(x, axis=1, keepdims=True).astype(jnp.int32)
    hit = pred == lbl_ref[...]
    v = jnp.where(conf > brow, 1.0, 0.0)
    v = jnp.where(li == _N_BINS + 2, conf, v)
    v = jnp.where(li == _N_BINS + 3,
                  jnp.where(hit, 1.0, 0.0), v)
    vb = v.astype(jnp.bfloat16)
    s = lax.dot_general(vb, vb, (((0,), (0,)), ((), ())),
                        preferred_element_type=jnp.float32)
    acc_ref[...] += s

    @pl.when(pid == _G - 1)
    def _fin():
        cum = acc_ref[_N_BINS + 1:_N_BINS + 4, :]
        per = cum[:, :_N_BINS] - cum[:, 1:_N_BINS + 1]
        cnt = per[0]
        sconf = per[1]
        sacc = per[2]
        has = cnt > 0.0
        denom = jnp.maximum(cnt, 1.0)
        acc_in = jnp.where(has, sacc / denom, 0.0)
        conf_in = jnp.where(has, sconf / denom, 0.0)
        prop = cnt * (1.0 / _N)
        ece = jnp.sum(jnp.where(has, jnp.abs(conf_in - acc_in) * prop, 0.0))
        ece_ref[...] = ece.reshape(1, 1)
        ys_ref[...] = acc_in.reshape(1, _N_BINS)


def kernel(logits, labels):
    lbl2 = labels.reshape(_N, 1)
    ece2, ys2 = pl.pallas_call(
        _body,
        grid=(_G,),
        in_specs=[
            pl.BlockSpec((_R, _C), lambda i: (i, 0)),
            pl.BlockSpec((_R, 1), lambda i: (i, 0)),
        ],
        out_specs=[
            pl.BlockSpec((1, 1), lambda i: (0, 0)),
            pl.BlockSpec((1, _N_BINS), lambda i: (0, 0)),
        ],
        out_shape=[
            jax.ShapeDtypeStruct((1, 1), jnp.float32),
            jax.ShapeDtypeStruct((1, _N_BINS), jnp.float32),
        ],
        scratch_shapes=[pltpu.VMEM((128, 128), jnp.float32)],
        compiler_params=pltpu.CompilerParams(
            dimension_semantics=("arbitrary",),
        ),
    )(logits, lbl2)
    return (ece2.reshape(1), ys2.reshape(_N_BINS))
